# baseline (speedup 1.0000x reference)
import functools

import jax
import jax.numpy as jnp
from jax import lax
from jax.experimental import pallas as pl
from jax.experimental.pallas import tpu as pltpu
from jax.experimental.pallas import tpu_sc as plsc

D = 1024
B = 4
S = 2048
NW = 32
ROWS_PER_W = (B * S) // NW
CH = 16
NCHUNK = ROWS_PER_W // CH
CHW = CH * D
WPB = NW // B


def _sc_body(x_hbm, pe_hbm, out_hbm, xbuf, pebuf, xsem, pesem, osem):
    wid = lax.axis_index("s") * 2 + lax.axis_index("c")
    base = pl.multiple_of(wid * (ROWS_PER_W * D), CHW)
    pe_base = pl.multiple_of((wid % WPB) * (ROWS_PER_W * D), CHW)

    def start(i, slot):
        off = base + i * CHW
        peoff = pe_base + i * CHW
        pltpu.make_async_copy(
            x_hbm.at[pl.ds(off, CHW)], xbuf.at[slot], xsem.at[slot]
        ).start()
        pltpu.make_async_copy(
            pe_hbm.at[pl.ds(peoff, CHW)], pebuf.at[slot], pesem.at[slot]
        ).start()

    def wait_in(i, slot):
        off = base + i * CHW
        peoff = pe_base + i * CHW
        pltpu.make_async_copy(
            x_hbm.at[pl.ds(off, CHW)], xbuf.at[slot], xsem.at[slot]
        ).wait()
        pltpu.make_async_copy(
            pe_hbm.at[pl.ds(peoff, CHW)], pebuf.at[slot], pesem.at[slot]
        ).wait()

    def start_out(i, slot):
        off = base + i * CHW
        pltpu.make_async_copy(
            xbuf.at[slot], out_hbm.at[pl.ds(off, CHW)], osem.at[slot]
        ).start()

    def wait_out(i, slot):
        off = base + i * CHW
        pltpu.make_async_copy(
            xbuf.at[slot], out_hbm.at[pl.ds(off, CHW)], osem.at[slot]
        ).wait()

    start(0, 0)
    for i in range(NCHUNK):
        slot = i % 2
        if i + 1 < NCHUNK:
            if i >= 1:
                wait_out(i - 1, 1 - slot)
            start(i + 1, 1 - slot)
        wait_in(i, slot)

        def add16(k, _):
            o = k * 16
            pebuf_v = pebuf[slot, pl.ds(o, 16)]
            xbuf[slot, pl.ds(o, 16)] += pebuf_v
            return 0

        lax.fori_loop(0, CHW // 16, add16, 0, unroll=8)
        start_out(i, slot)
    wait_out(NCHUNK - 2, NCHUNK % 2)
    wait_out(NCHUNK - 1, (NCHUNK - 1) % 2)


def kernel(x, pe):
    b, s, d = x.shape
    xf = x.reshape(b * s * d)
    pef = pe[:s].reshape(s * d)
    mesh = plsc.VectorSubcoreMesh(core_axis_name="c", subcore_axis_name="s")
    run = functools.partial(
        pl.kernel,
        mesh=mesh,
        out_type=jax.ShapeDtypeStruct((b * s * d,), x.dtype),
        scratch_types=[
            pltpu.VMEM((2, CHW), jnp.float32),
            pltpu.VMEM((2, CHW), jnp.float32),
            pltpu.SemaphoreType.DMA((2,)),
            pltpu.SemaphoreType.DMA((2,)),
            pltpu.SemaphoreType.DMA((2,)),
        ],
    )(_sc_body)
    out = run(xf, pef)
    return out.reshape(b, s, d)


# speedup vs baseline: 1.3887x; 1.3887x over previous
import functools

import jax
import jax.numpy as jnp
from jax import lax
from jax.experimental import pallas as pl
from jax.experimental.pallas import tpu as pltpu
from jax.experimental.pallas import tpu_sc as plsc

D = 1024
B = 4
S = 2048
NW = 32
ROWS_PER_W = (B * S) // NW
CH = 16
NCHUNK = ROWS_PER_W // CH
CHW = CH * D
WPB = NW // B


def _sc_body(x_hbm, pe_hbm, out_hbm, xbuf, pebuf, xsem, pesem, osem):
    wid = lax.axis_index("s") * 2 + lax.axis_index("c")
    base = pl.multiple_of(wid * (ROWS_PER_W * D), CHW)
    pe_base = pl.multiple_of((wid % WPB) * (ROWS_PER_W * D), CHW)

    def start(i, slot):
        off = base + i * CHW
        peoff = pe_base + i * CHW
        pltpu.make_async_copy(
            x_hbm.at[pl.ds(off, CHW)], xbuf.at[slot], xsem.at[slot]
        ).start()
        pltpu.make_async_copy(
            pe_hbm.at[pl.ds(peoff, CHW)], pebuf.at[slot], pesem.at[slot]
        ).start()

    def wait_in(i, slot):
        off = base + i * CHW
        peoff = pe_base + i * CHW
        pltpu.make_async_copy(
            x_hbm.at[pl.ds(off, CHW)], xbuf.at[slot], xsem.at[slot]
        ).wait()
        pltpu.make_async_copy(
            pe_hbm.at[pl.ds(peoff, CHW)], pebuf.at[slot], pesem.at[slot]
        ).wait()

    def start_out(i, slot):
        off = base + i * CHW
        pltpu.make_async_copy(
            xbuf.at[slot], out_hbm.at[pl.ds(off, CHW)], osem.at[slot]
        ).start()

    def wait_out(i, slot):
        off = base + i * CHW
        pltpu.make_async_copy(
            xbuf.at[slot], out_hbm.at[pl.ds(off, CHW)], osem.at[slot]
        ).wait()

    start(0, 0)
    for i in range(NCHUNK):
        slot = i % 2
        if i + 1 < NCHUNK:
            if i >= 1:
                wait_out(i - 1, 1 - slot)
            start(i + 1, 1 - slot)
        wait_in(i, slot)

        @plsc.parallel_loop(0, CHW // 16, 1, unroll=8)
        def _add16(k):
            o = k * 16
            plsc.addupdate(xbuf.at[slot, pl.ds(o, 16)], pebuf[slot, pl.ds(o, 16)])
        start_out(i, slot)
    wait_out(NCHUNK - 2, NCHUNK % 2)
    wait_out(NCHUNK - 1, (NCHUNK - 1) % 2)


def kernel(x, pe):
    b, s, d = x.shape
    xf = x.reshape(b * s * d)
    pef = pe[:s].reshape(s * d)
    mesh = plsc.VectorSubcoreMesh(core_axis_name="c", subcore_axis_name="s")
    run = functools.partial(
        pl.kernel,
        mesh=mesh,
        out_type=jax.ShapeDtypeStruct((b * s * d,), x.dtype),
        scratch_types=[
            pltpu.VMEM((2, CHW), jnp.float32),
            pltpu.VMEM((2, CHW), jnp.float32),
            pltpu.SemaphoreType.DMA((2,)),
            pltpu.SemaphoreType.DMA((2,)),
            pltpu.SemaphoreType.DMA((2,)),
        ],
    )(_sc_body)
    out = run(xf, pef)
    return out.reshape(b, s, d)


# speedup vs baseline: 1.4689x; 1.0578x over previous
import functools

import jax
import jax.numpy as jnp
from jax import lax
from jax.experimental import pallas as pl
from jax.experimental.pallas import tpu as pltpu
from jax.experimental.pallas import tpu_sc as plsc

D = 1024
B = 4
S = 2048
NW = 32
SROWS = S // NW
CH = 16
NCH_PER_B = SROWS // CH
CHW = CH * D
SW = SROWS * D


def _sc_body(x_hbm, pe_hbm, out_hbm, xbuf, pebuf, xsem, pesem, osem):
    wid = lax.axis_index("s") * 2 + lax.axis_index("c")
    pe_off = pl.multiple_of(wid * SW, CHW)

    pltpu.make_async_copy(pe_hbm.at[pl.ds(pe_off, SW)], pebuf, pesem).start()

    def x_off(b, i):
        return pl.multiple_of(b * (S * D) + pe_off + i * CHW, CHW)

    def start_in(b, i, slot):
        pltpu.make_async_copy(
            x_hbm.at[pl.ds(x_off(b, i), CHW)], xbuf.at[slot], xsem.at[slot]
        ).start()

    def wait_in(b, i, slot):
        pltpu.make_async_copy(
            x_hbm.at[pl.ds(x_off(b, i), CHW)], xbuf.at[slot], xsem.at[slot]
        ).wait()

    def start_out(b, i, slot):
        pltpu.make_async_copy(
            xbuf.at[slot], out_hbm.at[pl.ds(x_off(b, i), CHW)], osem.at[slot]
        ).start()

    def wait_out(b, i, slot):
        pltpu.make_async_copy(
            xbuf.at[slot], out_hbm.at[pl.ds(x_off(b, i), CHW)], osem.at[slot]
        ).wait()

    chunks = [(b, i) for b in range(B) for i in range(NCH_PER_B)]
    n = len(chunks)

    start_in(*chunks[0], 0)
    pltpu.make_async_copy(pe_hbm.at[pl.ds(pe_off, SW)], pebuf, pesem).wait()
    for k in range(n):
        b, i = chunks[k]
        slot = k % 2
        if k + 1 < n:
            if k >= 1:
                wait_out(*chunks[k - 1], 1 - slot)
            start_in(*chunks[k + 1], 1 - slot)
        wait_in(b, i, slot)

        pbase = i * CHW

        @plsc.parallel_loop(0, CHW // 16, 1, unroll=8)
        def _add16(j):
            o = j * 16
            plsc.addupdate(xbuf.at[slot, pl.ds(o, 16)], pebuf[pl.ds(pbase + o, 16)])

        start_out(b, i, slot)
    wait_out(*chunks[n - 2], n % 2)
    wait_out(*chunks[n - 1], (n - 1) % 2)


def kernel(x, pe):
    b, s, d = x.shape
    xf = x.reshape(b * s * d)
    pef = pe[:s].reshape(s * d)
    mesh = plsc.VectorSubcoreMesh(core_axis_name="c", subcore_axis_name="s")
    run = functools.partial(
        pl.kernel,
        mesh=mesh,
        out_type=jax.ShapeDtypeStruct((b * s * d,), x.dtype),
        scratch_types=[
            pltpu.VMEM((2, CHW), jnp.float32),
            pltpu.VMEM((SW,), jnp.float32),
            pltpu.SemaphoreType.DMA((2,)),
            pltpu.SemaphoreType.DMA,
            pltpu.SemaphoreType.DMA((2,)),
        ],
    )(_sc_body)
    out = run(xf, pef)
    return out.reshape(b, s, d)


# speedup vs baseline: 8.5247x; 5.8035x over previous
import jax
import jax.numpy as jnp
from jax.experimental import pallas as pl
from jax.experimental.pallas import tpu as pltpu


def _pe_add_kernel(x_ref, pe_ref, o_ref):
    o_ref[...] = x_ref[...] + pe_ref[...]


def kernel(x, pe):
    B, S, D = x.shape
    BS = 2048
    grid = (S // BS, B)
    return pl.pallas_call(
        _pe_add_kernel,
        grid=grid,
        in_specs=[
            pl.BlockSpec((1, BS, D), lambda s, b: (b, s, 0)),
            pl.BlockSpec((BS, D), lambda s, b: (s, 0)),
        ],
        out_specs=pl.BlockSpec((1, BS, D), lambda s, b: (b, s, 0)),
        out_shape=jax.ShapeDtypeStruct(x.shape, x.dtype),
        compiler_params=pltpu.CompilerParams(
            dimension_semantics=("parallel", "parallel"),
        ),
    )(x, pe[:S])


# speedup vs baseline: 8.6142x; 1.0105x over previous
import jax
import jax.numpy as jnp
from jax.experimental import pallas as pl
from jax.experimental.pallas import tpu as pltpu


def _pe_add_kernel(x_ref, pe_ref, o_ref):
    o_ref[...] = x_ref[...] + pe_ref[...]


def kernel(x, pe):
    B, S, D = x.shape
    xf = x.reshape(B * S, D)
    out = pl.pallas_call(
        _pe_add_kernel,
        grid=(B,),
        in_specs=[
            pl.BlockSpec((S, D), lambda b: (b, 0)),
            pl.BlockSpec((S, D), lambda b: (0, 0)),
        ],
        out_specs=pl.BlockSpec((S, D), lambda b: (b, 0)),
        out_shape=jax.ShapeDtypeStruct((B * S, D), x.dtype),
        compiler_params=pltpu.CompilerParams(
            dimension_semantics=("arbitrary",),
        ),
    )(xf, pe[:S])
    return out.reshape(B, S, D)


# speedup vs baseline: 8.6718x; 1.0067x over previous
import jax
import jax.numpy as jnp
from jax.experimental import pallas as pl
from jax.experimental.pallas import tpu as pltpu


def _pe_add_kernel(x_ref, pe_ref, o_ref):
    o_ref[...] = x_ref[...] + pe_ref[...]


def kernel(x, pe):
    B, S, D = x.shape
    BS = 2048
    grid = (S // BS, B)
    return pl.pallas_call(
        _pe_add_kernel,
        grid=grid,
        in_specs=[
            pl.BlockSpec((1, BS, D), lambda s, b: (b, s, 0)),
            pl.BlockSpec((BS, D), lambda s, b: (s, 0)),
        ],
        out_specs=pl.BlockSpec((1, BS, D), lambda s, b: (b, s, 0)),
        out_shape=jax.ShapeDtypeStruct(x.shape, x.dtype),
        compiler_params=pltpu.CompilerParams(
            dimension_semantics=("parallel", "parallel"),
        ),
    )(x, pe[:S])
